# baseline (speedup 1.0000x reference)
import jax
import jax.numpy as jnp
from jax import lax
from jax.experimental import pallas as pl
from jax.experimental.pallas import tpu as pltpu
from jax.experimental.pallas import tpu_sc as plsc

N = 2048
M = 64
PAD = 8
CH = 256
NCH = N // CH
CHW = CH * M
CPW = CHW + M


def _shift_body(v_ref, w_ref):
    v = v_ref[...]
    w_ref[pl.ds(0, N), :] = v - jnp.min(v, axis=1, keepdims=True) + 1.0
    w_ref[pl.ds(N, PAD), :] = jnp.ones((PAD, M), jnp.float32)


def _tc_shift(v):
    return pl.pallas_call(
        _shift_body,
        out_shape=jax.ShapeDtypeStruct((N + PAD, M), jnp.float32),
    )(v)


def _sc_scan_body(w_hbm, out_hbm, inb0, inb1, outb0, outb1, sin, sout):
    wid = lax.axis_index("s") * 2 + lax.axis_index("c")

    @pl.when(wid == 0)
    def _():
        ones = jnp.ones((16,), jnp.float32)
        lane = lax.iota(jnp.int32, 16)
        p8 = lane ^ 8
        p4 = lane ^ 4
        p2 = lane ^ 2
        p1 = lane ^ 1
        inbufs = (inb0, inb1)
        outbufs = (outb0, outb1)

        def make_row_step(inb, outb):
            def row_step(i, carry):
                c0, c1, c2, c3, z0, z1, z2, z3 = carry
                e0 = jnp.exp(z0)
                e1 = jnp.exp(z1)
                e2 = jnp.exp(z2)
                e3 = jnp.exp(z3)
                s = (e0 + e1) + (e2 + e3)
                s = s + s.at[p8].get(mode="promise_in_bounds")
                s = s + s.at[p4].get(mode="promise_in_bounds")
                s = s + s.at[p2].get(mode="promise_in_bounds")
                s = s + s.at[p1].get(mode="promise_in_bounds")
                inv = 1.0 / s
                base = i * M
                outb[pl.ds(base, 16)] = e0 * inv
                outb[pl.ds(base + 16, 16)] = e1 * inv
                outb[pl.ds(base + 32, 16)] = e2 * inv
                outb[pl.ds(base + 48, 16)] = e3 * inv
                a0 = inb[pl.ds(base + 64, 16)] * c0
                a1 = inb[pl.ds(base + 80, 16)] * c1
                a2 = inb[pl.ds(base + 96, 16)] * c2
                a3 = inb[pl.ds(base + 112, 16)] * c3
                b0 = a0 * e0
                b1 = a1 * e1
                b2 = a2 * e2
                b3 = a3 * e3
                return (c0 - (c0 * e0) * inv, c1 - (c1 * e1) * inv,
                        c2 - (c2 * e2) * inv, c3 - (c3 * e3) * inv,
                        a0 - b0 * inv, a1 - b1 * inv,
                        a2 - b2 * inv, a3 - b3 * inv)
            return row_step

        cp0 = pltpu.make_async_copy(w_hbm.at[pl.ds(0, CPW)], inb0, sin)
        cp0.start()
        cp0.wait()

        carry = (ones, ones, ones, ones,
                 inb0[pl.ds(0, 16)], inb0[pl.ds(16, 16)],
                 inb0[pl.ds(32, 16)], inb0[pl.ds(48, 16)])
        for g in range(NCH):
            inb = inbufs[g % 2]
            outb = outbufs[g % 2]
            if g + 1 < NCH:
                nxt = pltpu.make_async_copy(
                    w_hbm.at[pl.ds((g + 1) * CHW, CPW)], inbufs[(g + 1) % 2], sin)
                nxt.start()
            if g >= 2:
                pltpu.make_async_copy(
                    outb, out_hbm.at[pl.ds((g - 2) * CHW, CHW)], sout).wait()
            carry = lax.fori_loop(0, CH, make_row_step(inb, outb), carry,
                                  unroll=4)
            st = pltpu.make_async_copy(outb, out_hbm.at[pl.ds(g * CHW, CHW)], sout)
            st.start()
            if g + 1 < NCH:
                nxt.wait()
        pltpu.make_async_copy(
            outbufs[(NCH - 2) % 2], out_hbm.at[pl.ds((NCH - 2) * CHW, CHW)], sout).wait()
        pltpu.make_async_copy(
            outbufs[(NCH - 1) % 2], out_hbm.at[pl.ds((NCH - 1) * CHW, CHW)], sout).wait()


_sc_scan = pl.kernel(
    _sc_scan_body,
    out_type=jax.ShapeDtypeStruct((N * M,), jnp.float32),
    mesh=plsc.VectorSubcoreMesh(core_axis_name="c", subcore_axis_name="s"),
    scratch_types=[
        pltpu.VMEM((CPW,), jnp.float32),
        pltpu.VMEM((CPW,), jnp.float32),
        pltpu.VMEM((CHW,), jnp.float32),
        pltpu.VMEM((CHW,), jnp.float32),
        pltpu.SemaphoreType.DMA,
        pltpu.SemaphoreType.DMA,
    ],
)


def kernel(V):
    w = _tc_shift(V).reshape((N + PAD) * M)
    return _sc_scan(w).reshape(N, M)


# speedup vs baseline: 1.0822x; 1.0822x over previous
import functools

import jax
import jax.numpy as jnp
from jax import lax
from jax.experimental import pallas as pl
from jax.experimental.pallas import tpu as pltpu
from jax.experimental.pallas import tpu_sc as plsc

N = 2048
M = 64
CH = 256
NCH = N // CH
CHW = CH * M


def _shift_body(v_ref, w_ref):
    v = v_ref[...]
    w_ref[...] = v - jnp.min(v, axis=1, keepdims=True) + 1.0


def _tc_shift(v):
    return pl.pallas_call(
        _shift_body,
        out_shape=jax.ShapeDtypeStruct((N, M), jnp.float32),
    )(v)


def _sc_scan_body(w_hbm, out_hbm, inb0, inb1, outb0, outb1, sin, sout):
    wid = lax.axis_index("s") * 2 + lax.axis_index("c")

    @pl.when(wid == 0)
    def _():
        ones = jnp.ones((16,), jnp.float32)
        c = (ones, ones, ones, ones)
        lane = lax.iota(jnp.int32, 16)
        p8 = lane ^ 8
        p4 = lane ^ 4
        p2 = lane ^ 2
        p1 = lane ^ 1

        inbufs = (inb0, inb1)
        outbufs = (outb0, outb1)

        def make_row_step(inb, outb):
            def row_step(i, c4):
                c0, c1, c2, c3 = c4
                base = i * M
                e0 = jnp.exp(inb[pl.ds(base, 16)] * c0)
                e1 = jnp.exp(inb[pl.ds(base + 16, 16)] * c1)
                e2 = jnp.exp(inb[pl.ds(base + 32, 16)] * c2)
                e3 = jnp.exp(inb[pl.ds(base + 48, 16)] * c3)
                s = (e0 + e1) + (e2 + e3)
                s = s + s.at[p8].get(mode="promise_in_bounds")
                s = s + s.at[p4].get(mode="promise_in_bounds")
                s = s + s.at[p2].get(mode="promise_in_bounds")
                s = s + s.at[p1].get(mode="promise_in_bounds")
                inv = 1.0 / s
                y0 = e0 * inv
                y1 = e1 * inv
                y2 = e2 * inv
                y3 = e3 * inv
                outb[pl.ds(base, 16)] = y0
                outb[pl.ds(base + 16, 16)] = y1
                outb[pl.ds(base + 32, 16)] = y2
                outb[pl.ds(base + 48, 16)] = y3
                return (c0 - c0 * y0, c1 - c1 * y1, c2 - c2 * y2, c3 - c3 * y3)
            return row_step

        cp0 = pltpu.make_async_copy(w_hbm.at[pl.ds(0, CHW)], inb0, sin)
        cp0.start()
        cp0.wait()

        for g in range(NCH):
            inb = inbufs[g % 2]
            outb = outbufs[g % 2]
            if g + 1 < NCH:
                nxt = pltpu.make_async_copy(
                    w_hbm.at[pl.ds((g + 1) * CHW, CHW)], inbufs[(g + 1) % 2], sin)
                nxt.start()
            if g >= 2:
                pltpu.make_async_copy(
                    outb, out_hbm.at[pl.ds((g - 2) * CHW, CHW)], sout).wait()
            c = lax.fori_loop(0, CH, make_row_step(inb, outb), c, unroll=4)
            st = pltpu.make_async_copy(outb, out_hbm.at[pl.ds(g * CHW, CHW)], sout)
            st.start()
            if g + 1 < NCH:
                nxt.wait()
        pltpu.make_async_copy(
            outbufs[(NCH - 2) % 2], out_hbm.at[pl.ds((NCH - 2) * CHW, CHW)], sout).wait()
        pltpu.make_async_copy(
            outbufs[(NCH - 1) % 2], out_hbm.at[pl.ds((NCH - 1) * CHW, CHW)], sout).wait()


_sc_scan = pl.kernel(
    _sc_scan_body,
    out_type=jax.ShapeDtypeStruct((N * M,), jnp.float32),
    mesh=plsc.VectorSubcoreMesh(core_axis_name="c", subcore_axis_name="s", num_cores=1),
    scratch_types=[
        pltpu.VMEM((CHW,), jnp.float32),
        pltpu.VMEM((CHW,), jnp.float32),
        pltpu.VMEM((CHW,), jnp.float32),
        pltpu.VMEM((CHW,), jnp.float32),
        pltpu.SemaphoreType.DMA,
        pltpu.SemaphoreType.DMA,
    ],
)


def kernel(V):
    w = _tc_shift(V).reshape(N * M)
    return _sc_scan(w).reshape(N, M)


# speedup vs baseline: 1.0891x; 1.0064x over previous
import functools

import jax
import jax.numpy as jnp
from jax import lax
from jax.experimental import pallas as pl
from jax.experimental.pallas import tpu as pltpu
from jax.experimental.pallas import tpu_sc as plsc

N = 2048
M = 64
CH = 256
NCH = N // CH
CHW = CH * M


def _shift_body(v_ref, w_ref):
    v = v_ref[...]
    w_ref[...] = v - jnp.min(v, axis=1, keepdims=True) + 1.0


def _tc_shift(v):
    return pl.pallas_call(
        _shift_body,
        out_shape=jax.ShapeDtypeStruct((N, M), jnp.float32),
    )(v)


def _sc_scan_body(w_hbm, out_hbm, inb0, inb1, outb0, outb1, sin, sout):
    wid = lax.axis_index("s") * 2 + lax.axis_index("c")

    @pl.when(wid == 0)
    def _():
        ones = jnp.ones((16,), jnp.float32)
        c = (ones, ones, ones, ones)
        lane = lax.iota(jnp.int32, 16)
        p8 = lane ^ 8
        p4 = lane ^ 4
        p2 = lane ^ 2
        p1 = lane ^ 1

        inbufs = (inb0, inb1)
        outbufs = (outb0, outb1)

        def make_row_step(inb, outb):
            def row_step(i, c4):
                c0, c1, c2, c3 = c4
                e0 = jnp.exp(inb[i, pl.ds(0, 16)] * c0)
                e1 = jnp.exp(inb[i, pl.ds(16, 16)] * c1)
                e2 = jnp.exp(inb[i, pl.ds(32, 16)] * c2)
                e3 = jnp.exp(inb[i, pl.ds(48, 16)] * c3)
                s = (e0 + e1) + (e2 + e3)
                s = s + s.at[p8].get(mode="promise_in_bounds")
                s = s + s.at[p4].get(mode="promise_in_bounds")
                s = s + s.at[p2].get(mode="promise_in_bounds")
                s = s + s.at[p1].get(mode="promise_in_bounds")
                inv = 1.0 / s
                y0 = e0 * inv
                y1 = e1 * inv
                y2 = e2 * inv
                y3 = e3 * inv
                outb[i, pl.ds(0, 16)] = y0
                outb[i, pl.ds(16, 16)] = y1
                outb[i, pl.ds(32, 16)] = y2
                outb[i, pl.ds(48, 16)] = y3
                return (c0 - c0 * y0, c1 - c1 * y1, c2 - c2 * y2, c3 - c3 * y3)
            return row_step

        cp0 = pltpu.make_async_copy(w_hbm.at[pl.ds(0, CH)], inb0, sin)
        cp0.start()
        cp0.wait()

        for g in range(NCH):
            inb = inbufs[g % 2]
            outb = outbufs[g % 2]
            if g + 1 < NCH:
                nxt = pltpu.make_async_copy(
                    w_hbm.at[pl.ds((g + 1) * CH, CH)], inbufs[(g + 1) % 2], sin)
                nxt.start()
            if g >= 2:
                pltpu.make_async_copy(
                    outb, out_hbm.at[pl.ds((g - 2) * CH, CH)], sout).wait()
            c = lax.fori_loop(0, CH, make_row_step(inb, outb), c, unroll=4)
            st = pltpu.make_async_copy(outb, out_hbm.at[pl.ds(g * CH, CH)], sout)
            st.start()
            if g + 1 < NCH:
                nxt.wait()
        pltpu.make_async_copy(
            outbufs[(NCH - 2) % 2], out_hbm.at[pl.ds((NCH - 2) * CH, CH)], sout).wait()
        pltpu.make_async_copy(
            outbufs[(NCH - 1) % 2], out_hbm.at[pl.ds((NCH - 1) * CH, CH)], sout).wait()


_sc_scan = pl.kernel(
    _sc_scan_body,
    out_type=jax.ShapeDtypeStruct((N, M), jnp.float32),
    mesh=plsc.VectorSubcoreMesh(core_axis_name="c", subcore_axis_name="s", num_cores=1),
    scratch_types=[
        pltpu.VMEM((CH, M), jnp.float32),
        pltpu.VMEM((CH, M), jnp.float32),
        pltpu.VMEM((CH, M), jnp.float32),
        pltpu.VMEM((CH, M), jnp.float32),
        pltpu.SemaphoreType.DMA,
        pltpu.SemaphoreType.DMA,
    ],
)


def kernel(V):
    return _sc_scan(_tc_shift(V))


# speedup vs baseline: 1.1024x; 1.0122x over previous
import functools

import jax
import jax.numpy as jnp
from jax import lax
from jax.experimental import pallas as pl
from jax.experimental.pallas import tpu as pltpu
from jax.experimental.pallas import tpu_sc as plsc

N = 2048
M = 64
CH = 256
NCH = N // CH
CHW = CH * M


def _shift_body(v_ref, w_ref):
    v = v_ref[...]
    w_ref[...] = v - jnp.min(v, axis=1, keepdims=True) + 1.0


def _tc_shift(v):
    return pl.pallas_call(
        _shift_body,
        out_shape=jax.ShapeDtypeStruct((N, M), jnp.float32),
    )(v)


def _sc_scan_body(w_hbm, out_hbm, inb0, inb1, outb0, outb1, sin, sout):
    wid = lax.axis_index("s") * 2 + lax.axis_index("c")

    @pl.when(wid == 0)
    def _():
        ones = jnp.ones((16,), jnp.float32)
        c = (ones, ones, ones, ones)
        lane = lax.iota(jnp.int32, 16)
        p8 = lane ^ 8
        p4 = lane ^ 4
        p2 = lane ^ 2
        p1 = lane ^ 1

        inbufs = (inb0, inb1)
        outbufs = (outb0, outb1)

        def make_row_step(inb, outb):
            def row_step(i, c4):
                c0, c1, c2, c3 = c4
                e0 = jnp.exp(inb[i, pl.ds(0, 16)] * c0)
                e1 = jnp.exp(inb[i, pl.ds(16, 16)] * c1)
                e2 = jnp.exp(inb[i, pl.ds(32, 16)] * c2)
                e3 = jnp.exp(inb[i, pl.ds(48, 16)] * c3)
                s = (e0 + e1) + (e2 + e3)
                s = s + s.at[p8].get(mode="promise_in_bounds")
                s = s + s.at[p4].get(mode="promise_in_bounds")
                s = s + s.at[p2].get(mode="promise_in_bounds")
                s = s + s.at[p1].get(mode="promise_in_bounds")
                inv = 1.0 / s
                y0 = e0 * inv
                y1 = e1 * inv
                y2 = e2 * inv
                y3 = e3 * inv
                outb[i, pl.ds(0, 16)] = y0
                outb[i, pl.ds(16, 16)] = y1
                outb[i, pl.ds(32, 16)] = y2
                outb[i, pl.ds(48, 16)] = y3
                return (c0 - c0 * y0, c1 - c1 * y1, c2 - c2 * y2, c3 - c3 * y3)
            return row_step

        cp0 = pltpu.make_async_copy(w_hbm.at[pl.ds(0, CH)], inb0, sin)
        cp0.start()
        cp0.wait()

        for g in range(NCH):
            inb = inbufs[g % 2]
            outb = outbufs[g % 2]
            if g + 1 < NCH:
                nxt = pltpu.make_async_copy(
                    w_hbm.at[pl.ds((g + 1) * CH, CH)], inbufs[(g + 1) % 2], sin)
                nxt.start()
            if g >= 2:
                pltpu.make_async_copy(
                    outb, out_hbm.at[pl.ds((g - 2) * CH, CH)], sout).wait()
            c = lax.fori_loop(0, CH, make_row_step(inb, outb), c, unroll=8)
            st = pltpu.make_async_copy(outb, out_hbm.at[pl.ds(g * CH, CH)], sout)
            st.start()
            if g + 1 < NCH:
                nxt.wait()
        pltpu.make_async_copy(
            outbufs[(NCH - 2) % 2], out_hbm.at[pl.ds((NCH - 2) * CH, CH)], sout).wait()
        pltpu.make_async_copy(
            outbufs[(NCH - 1) % 2], out_hbm.at[pl.ds((NCH - 1) * CH, CH)], sout).wait()


_sc_scan = pl.kernel(
    _sc_scan_body,
    out_type=jax.ShapeDtypeStruct((N, M), jnp.float32),
    mesh=plsc.VectorSubcoreMesh(core_axis_name="c", subcore_axis_name="s", num_cores=1),
    scratch_types=[
        pltpu.VMEM((CH, M), jnp.float32),
        pltpu.VMEM((CH, M), jnp.float32),
        pltpu.VMEM((CH, M), jnp.float32),
        pltpu.VMEM((CH, M), jnp.float32),
        pltpu.SemaphoreType.DMA,
        pltpu.SemaphoreType.DMA,
    ],
)


def kernel(V):
    return _sc_scan(_tc_shift(V))


# speedup vs baseline: 1.1746x; 1.0656x over previous
import functools

import jax
import jax.numpy as jnp
from jax import lax
from jax.experimental import pallas as pl
from jax.experimental.pallas import tpu as pltpu
from jax.experimental.pallas import tpu_sc as plsc

N = 2048
M = 64
CH = 256
NCH = N // CH
CHW = CH * M


def _shift_body(v_ref, w_ref):
    v = v_ref[...]
    w_ref[...] = v - jnp.min(v, axis=1, keepdims=True) + 1.0


def _tc_shift(v):
    return pl.pallas_call(
        _shift_body,
        out_shape=jax.ShapeDtypeStruct((N, M), jnp.float32),
    )(v)


def _sc_scan_body(w_hbm, out_hbm, inb0, inb1, outb0, outb1, sin, sout):
    wid = lax.axis_index("s") * 2 + lax.axis_index("c")

    @pl.when(wid == 0)
    def _():
        ones = jnp.ones((16,), jnp.float32)
        c = (ones, ones, ones, ones)
        lane = lax.iota(jnp.int32, 16)
        p8 = lane ^ 8
        p4 = lane ^ 4
        p2 = lane ^ 2
        p1 = lane ^ 1

        inbufs = (inb0, inb1)
        outbufs = (outb0, outb1)

        def make_row_step(inb, outb):
            def row_step(i, c4):
                c0, c1, c2, c3 = c4
                e0 = jnp.exp(inb[i, pl.ds(0, 16)] * c0)
                e1 = jnp.exp(inb[i, pl.ds(16, 16)] * c1)
                e2 = jnp.exp(inb[i, pl.ds(32, 16)] * c2)
                e3 = jnp.exp(inb[i, pl.ds(48, 16)] * c3)
                s = (e0 + e1) + (e2 + e3)
                s = s + s.at[p8].get(mode="promise_in_bounds")
                s = s + s.at[p4].get(mode="promise_in_bounds")
                s = s + s.at[p2].get(mode="promise_in_bounds")
                s = s + s.at[p1].get(mode="promise_in_bounds")
                inv = 1.0 / s
                y0 = e0 * inv
                y1 = e1 * inv
                y2 = e2 * inv
                y3 = e3 * inv
                outb[i, pl.ds(0, 16)] = y0
                outb[i, pl.ds(16, 16)] = y1
                outb[i, pl.ds(32, 16)] = y2
                outb[i, pl.ds(48, 16)] = y3
                return (c0 - c0 * y0, c1 - c1 * y1, c2 - c2 * y2, c3 - c3 * y3)
            return row_step

        cp0 = pltpu.make_async_copy(w_hbm.at[pl.ds(0, CH)], inb0, sin)
        cp0.start()
        cp0.wait()

        for g in range(NCH):
            inb = inbufs[g % 2]
            outb = outbufs[g % 2]
            if g + 1 < NCH:
                nxt = pltpu.make_async_copy(
                    w_hbm.at[pl.ds((g + 1) * CH, CH)], inbufs[(g + 1) % 2], sin)
                nxt.start()
            if g >= 2:
                pltpu.make_async_copy(
                    outb, out_hbm.at[pl.ds((g - 2) * CH, CH)], sout).wait()
            c = plsc.parallel_loop(0, CH, 1, unroll=8, carry=c)(
                make_row_step(inb, outb))
            st = pltpu.make_async_copy(outb, out_hbm.at[pl.ds(g * CH, CH)], sout)
            st.start()
            if g + 1 < NCH:
                nxt.wait()
        pltpu.make_async_copy(
            outbufs[(NCH - 2) % 2], out_hbm.at[pl.ds((NCH - 2) * CH, CH)], sout).wait()
        pltpu.make_async_copy(
            outbufs[(NCH - 1) % 2], out_hbm.at[pl.ds((NCH - 1) * CH, CH)], sout).wait()


_sc_scan = pl.kernel(
    _sc_scan_body,
    out_type=jax.ShapeDtypeStruct((N, M), jnp.float32),
    mesh=plsc.VectorSubcoreMesh(core_axis_name="c", subcore_axis_name="s", num_cores=1),
    scratch_types=[
        pltpu.VMEM((CH, M), jnp.float32),
        pltpu.VMEM((CH, M), jnp.float32),
        pltpu.VMEM((CH, M), jnp.float32),
        pltpu.VMEM((CH, M), jnp.float32),
        pltpu.SemaphoreType.DMA,
        pltpu.SemaphoreType.DMA,
    ],
)


def kernel(V):
    return _sc_scan(_tc_shift(V))
